# baseline (speedup 1.0000x reference)
import jax
import jax.numpy as jnp
from jax import lax
from jax.experimental import pallas as pl
from jax.experimental.pallas import tpu as pltpu
from jax.experimental.pallas import tpu_sc as plsc

N_TABLES = 26
NUM_ROWS = 100000
DIM = 64
BATCH = 1024
BAG = 20

NW = 32
BAGS_PW = BATCH // NW
IDX_PW = BAGS_PW * BAG
NCHUNK = IDX_PW // 128
NSPLIT = 4
ACC_PW = NSPLIT * BAGS_PW


def _body(idx_hbm, bag_hbm, wt_hbm, out_hbm,
          idx_v, bag_v, rows_v, accv, out_v, acc_sh, zero_v, sem):
    cid = lax.axis_index("c")
    sid = lax.axis_index("s")
    wid = sid * 2 + cid

    pltpu.sync_copy(bag_hbm, bag_v)
    slab = (sid * ACC_PW).astype(jnp.int32)
    for j in range(NCHUNK):
        for c in range(8):
            bag_v[j, pl.ds(c * 16, 16)] = bag_v[j, pl.ds(c * 16, 16)] + slab
    z16 = jnp.zeros((16,), jnp.float32)
    for i in range(ACC_PW):
        for c in range(DIM // 16):
            zero_v[i, pl.ds(c * 16, 16)] = z16

    def t_step(t, carry):
        base = pl.multiple_of(t * (BATCH * BAG) + wid * IDX_PW, 8)
        pltpu.sync_copy(idx_hbm.at[pl.ds(base, IDX_PW)], idx_v)
        off = (t * NUM_ROWS).astype(jnp.int32)
        for c in range(IDX_PW // 16):
            idx_v[pl.ds(c * 16, 16)] = idx_v[pl.ds(c * 16, 16)] + off
        pltpu.sync_copy(zero_v, acc_sh.at[pl.ds(sid * ACC_PW, ACC_PW)])
        handles = [
            pltpu.async_copy(wt_hbm.at[idx_v.at[pl.ds(j * 128, 128)]],
                             rows_v.at[pl.ds(j * 128, 128)], sem)
            for j in range(NCHUNK)
        ]
        for h in handles:
            h.wait()
        for j in range(NCHUNK):
            pltpu.sync_copy(rows_v.at[pl.ds(j * 128, 128)],
                            acc_sh.at[bag_v.at[j]], add=True)
        pltpu.sync_copy(acc_sh.at[pl.ds(sid * ACC_PW, ACC_PW)], accv)
        for b in range(BAGS_PW):
            for c in range(DIM // 16):
                s = pl.ds(c * 16, 16)
                acc = accv[b, s]
                for m in range(1, NSPLIT):
                    acc = acc + accv[m * BAGS_PW + b, s]
                out_v[b, s] = acc
        pltpu.sync_copy(out_v,
                        out_hbm.at[t].at[pl.ds(wid * BAGS_PW, BAGS_PW)])
        return carry

    lax.fori_loop(0, N_TABLES, t_step, 0)


@jax.jit
def _run(idx_pm, bag_ids, wt_merged):
    mesh = plsc.VectorSubcoreMesh(core_axis_name="c", subcore_axis_name="s")
    f = pl.kernel(
        _body,
        out_type=jax.ShapeDtypeStruct((N_TABLES, BATCH, DIM), jnp.float32),
        mesh=mesh,
        scratch_types=[
            pltpu.VMEM((IDX_PW,), jnp.int32),
            pltpu.VMEM((NCHUNK, 128), jnp.int32),
            pltpu.VMEM((IDX_PW, DIM), jnp.float32),
            pltpu.VMEM((ACC_PW, DIM), jnp.float32),
            pltpu.VMEM((BAGS_PW, DIM), jnp.float32),
            pltpu.VMEM_SHARED((16 * ACC_PW, DIM), jnp.float32),
            pltpu.VMEM((ACC_PW, DIM), jnp.float32),
            pltpu.SemaphoreType.DMA,
        ],
        compiler_params=pltpu.CompilerParams(use_tc_tiling_on_sc=False),
    )
    return f(idx_pm, bag_ids, wt_merged)


def kernel(indices, weights):
    idx_pm = (indices.astype(jnp.int32)
              .reshape(N_TABLES, NW, BAGS_PW, BAG)
              .transpose(0, 1, 3, 2)
              .reshape(N_TABLES * BATCH * BAG))
    wt_merged = weights.reshape(N_TABLES * NUM_ROWS, DIM)
    p = jnp.arange(IDX_PW, dtype=jnp.int32)
    bag_ids = (((p // BAGS_PW) % NSPLIT) * BAGS_PW
               + (p % BAGS_PW)).reshape(NCHUNK, 128)
    return _run(idx_pm, bag_ids, wt_merged)


# speedup vs baseline: 1.0527x; 1.0527x over previous
import jax
import jax.numpy as jnp
from jax import lax
from jax.experimental import pallas as pl
from jax.experimental.pallas import tpu as pltpu
from jax.experimental.pallas import tpu_sc as plsc

N_TABLES = 26
NUM_ROWS = 100000
DIM = 64
BATCH = 1024
BAG = 20

NW = 32
BAGS_PW = BATCH // NW
IDX_PW = BAGS_PW * BAG
NCHUNK = IDX_PW // 128
IDX_ALL = N_TABLES * IDX_PW


def _body(idx_hbm, wt_hbm, out_hbm, idx_v, rows_v, out_v, sem):
    cid = lax.axis_index("c")
    sid = lax.axis_index("s")
    wid = sid * 2 + cid

    ibase = pl.multiple_of(wid * IDX_ALL, 8)
    pltpu.sync_copy(idx_hbm.at[pl.ds(ibase, IDX_ALL)], idx_v)

    def lin_step(t, carry):
        off = (t * NUM_ROWS).astype(jnp.int32)
        tb = t * IDX_PW
        for c in range(IDX_PW // 16):
            s = pl.ds(tb + c * 16, 16)
            idx_v[s] = idx_v[s] + off
        return carry

    lax.fori_loop(0, N_TABLES, lin_step, 0)

    def fire(t, par):
        for j in range(NCHUNK):
            pltpu.async_copy(
                wt_hbm.at[idx_v.at[pl.ds(t * IDX_PW + j * 128, 128)]],
                rows_v.at[pl.ds(par * IDX_PW + j * 128, 128)], sem)

    def drain(t, par):
        for j in range(NCHUNK):
            pltpu.make_async_copy(
                wt_hbm.at[idx_v.at[pl.ds(t * IDX_PW + j * 128, 128)]],
                rows_v.at[pl.ds(par * IDX_PW + j * 128, 128)], sem).wait()

    fire(0, 0)

    def t_step(t, carry):
        par = t % 2
        drain(t, par)

        @pl.when(t + 1 < N_TABLES)
        def _():
            fire(t + 1, 1 - par)

        def bag_step(b, carry2):
            rb = par * IDX_PW + b * BAG
            for c in range(DIM // 16):
                s = pl.ds(c * 16, 16)
                acc = rows_v[rb, s]
                for k in range(1, BAG):
                    acc = acc + rows_v[rb + k, s]
                out_v[b, s] = acc
            return carry2

        lax.fori_loop(0, BAGS_PW, bag_step, 0)
        pltpu.sync_copy(out_v,
                        out_hbm.at[t].at[pl.ds(wid * BAGS_PW, BAGS_PW)])
        return carry

    lax.fori_loop(0, N_TABLES, t_step, 0)


@jax.jit
def _run(idx_wm, wt_merged):
    mesh = plsc.VectorSubcoreMesh(core_axis_name="c", subcore_axis_name="s")
    f = pl.kernel(
        _body,
        out_type=jax.ShapeDtypeStruct((N_TABLES, BATCH, DIM), jnp.float32),
        mesh=mesh,
        scratch_types=[
            pltpu.VMEM((IDX_ALL,), jnp.int32),
            pltpu.VMEM((2 * IDX_PW, DIM), jnp.float32),
            pltpu.VMEM((BAGS_PW, DIM), jnp.float32),
            pltpu.SemaphoreType.DMA,
        ],
        compiler_params=pltpu.CompilerParams(use_tc_tiling_on_sc=False),
    )
    return f(idx_wm, wt_merged)


def kernel(indices, weights):
    idx_wm = (indices.astype(jnp.int32)
              .reshape(N_TABLES, NW, BAGS_PW * BAG)
              .transpose(1, 0, 2)
              .reshape(N_TABLES * BATCH * BAG))
    wt_merged = weights.reshape(N_TABLES * NUM_ROWS, DIM)
    return _run(idx_wm, wt_merged)
